# baseline (speedup 1.0000x reference)
import functools
import jax
import jax.numpy as jnp
from jax import lax
from jax.experimental import pallas as pl
from jax.experimental.pallas import tpu as pltpu
from jax.experimental.pallas import tpu_sc as plsc

FN = 64
L = 64
RBLK = 256
NW = 32
CH = 128


def _tc_body(depth_ref, dens_ref, cdf_ref, a_ref, s_ref):
    R = depth_ref.shape[1]
    depth = depth_ref[...]
    dens = dens_ref[...]

    delta = jnp.concatenate(
        [depth[1:L] - depth[0:L - 1], jnp.full((1, R), 1e10, jnp.float32)], axis=0)
    neg_x = (-jnp.maximum(dens, 0.0)) * delta
    E = jnp.exp(neg_x)
    alpha = 1.0 - E
    terms = (1.0 - alpha) + 1e-10

    rows = [jnp.ones((1, R), jnp.float32)]
    for i in range(1, L):
        rows.append(rows[-1] * terms[i - 1:i])
    trans = jnp.concatenate(rows, axis=0)

    wp = alpha * trans + 1e-5

    P = wp[1:9]
    for k in range(1, 7):
        P = P + wp[1 + 8 * k: 9 + 8 * k]
    P = P + jnp.concatenate([wp[57:64], jnp.zeros((1, R), jnp.float32)], axis=0)
    W = (((P[0:1] + P[4:5]) + (P[2:3] + P[6:7]))
         + ((P[1:2] + P[5:6]) + (P[3:4] + P[7:8])))

    inv_w = 1.0 / W
    pdf = wp[1:L] * inv_w

    crows = [jnp.zeros((1, R), jnp.float32), pdf[0:1]]
    for i in range(1, L - 1):
        crows.append(crows[-1] + pdf[i:i + 1])
    cdf_s = jnp.concatenate(crows, axis=0)

    dn = cdf_s[1:L] - cdf_s[0:L - 1]
    dn = jnp.where(dn < 1e-5, jnp.ones_like(dn), dn)
    S_lv = (depth[1:L] - depth[0:L - 1]) / dn
    A_lv = depth[0:L - 1] - S_lv * cdf_s[0:L - 1]

    zrow = jnp.zeros((1, R), jnp.float32)
    cdf_ref[...] = cdf_s
    a_ref[...] = jnp.concatenate([A_lv, zrow], axis=0)
    s_ref[...] = jnp.concatenate([S_lv, zrow], axis=0)


def _tc_stage(depth_t, dens_t, n):
    grid = n // RBLK
    return pl.pallas_call(
        _tc_body,
        grid=(grid,),
        in_specs=[
            pl.BlockSpec((L, RBLK), lambda i: (0, i)),
            pl.BlockSpec((L, RBLK), lambda i: (0, i)),
        ],
        out_specs=[
            pl.BlockSpec((L, RBLK), lambda i: (0, i)),
            pl.BlockSpec((L, RBLK), lambda i: (0, i)),
            pl.BlockSpec((L, RBLK), lambda i: (0, i)),
        ],
        out_shape=[
            jax.ShapeDtypeStruct((L, n), jnp.float32),
            jax.ShapeDtypeStruct((L, n), jnp.float32),
            jax.ShapeDtypeStruct((L, n), jnp.float32),
        ],
        compiler_params=pltpu.CompilerParams(
            dimension_semantics=("arbitrary",),
        ),
    )(depth_t, dens_t)


def _make_sc_stage(n):
    mesh = plsc.VectorSubcoreMesh(core_axis_name="c", subcore_axis_name="s")
    per_w = n // NW

    @functools.partial(
        pl.kernel, mesh=mesh,
        compiler_params=pltpu.CompilerParams(needs_layout_passes=False),
        out_type=[
            jax.ShapeDtypeStruct((n * 64,), jnp.float32),
            jax.ShapeDtypeStruct((n * 192,), jnp.float32),
        ],
        scratch_types=[
            pltpu.VMEM((64,), jnp.float32),
            pltpu.VMEM((CH * 64,), jnp.float32),
            pltpu.VMEM((CH * 64,), jnp.float32),
            pltpu.VMEM((CH * 64,), jnp.float32),
            pltpu.VMEM((CH * 6 + 16,), jnp.float32),
            pltpu.VMEM((128,), jnp.float32),
            pltpu.VMEM((CH * 64,), jnp.float32),
            pltpu.VMEM((CH * 192,), jnp.float32),
        ],
    )
    def sc_stage(cdfs_hbm, alv_hbm, slv_hbm, rays_hbm, u_hbm,
                 t_hbm, fsp_hbm,
                 u_v, cdf_v, alv_v, slv_v, rays_v, h_v, t_v, fsp_v):
        wid = lax.axis_index("s") * 2 + lax.axis_index("c")
        base_ray = wid * per_w
        pltpu.sync_copy(u_hbm, u_v)
        lane = lax.iota(jnp.int32, 16)
        lane3 = lane * 3
        ones16 = jnp.ones((16,), jnp.float32)
        zeros16 = jnp.zeros((16,), jnp.float32)

        @pl.loop(0, per_w // CH)
        def _chunk(ci):
            cbase = base_ray + ci * CH
            pltpu.sync_copy(cdfs_hbm.at[pl.ds(cbase * 64, CH * 64)], cdf_v)
            pltpu.sync_copy(alv_hbm.at[pl.ds(cbase * 64, CH * 64)], alv_v)
            pltpu.sync_copy(slv_hbm.at[pl.ds(cbase * 64, CH * 64)], slv_v)
            pltpu.sync_copy(rays_hbm.at[pl.ds(cbase * 6, CH * 6)],
                            rays_v.at[pl.ds(0, CH * 6)])

            @pl.loop(0, CH)
            def _ray(r):
                b = r * 64
                rv = rays_v[pl.ds(r * 6, 16)]
                o0 = rv[0]
                o1 = rv[1]
                o2 = rv[2]
                d0 = rv[3]
                d1 = rv[4]
                d2 = rv[5]

                for k in range(4):
                    uu = u_v[pl.ds(16 * k, 16)]
                    cnt = jnp.zeros((16,), jnp.int32)
                    for step in (32, 16, 8, 4, 2, 1):
                        c = plsc.load_gather(
                            cdf_v, [b + cnt + (step - 1)])
                        cnt = jnp.where(c < uu, cnt + step, cnt)
                    ctop = plsc.load_gather(cdf_v, [lane * 0 + (b + 63)])
                    full = (cnt == 63) & (ctop < uu)
                    above = jnp.maximum(cnt, 1)
                    above = jnp.where(full, 1, above)
                    idx = (above - 1) + b
                    A = plsc.load_gather(alv_v, [idx])
                    S = plsc.load_gather(slv_v, [idx])
                    t = A + S * uu
                    t_v[pl.ds(b + 16 * k, 16)] = t
                    fb = r * 192 + 48 * k
                    plsc.store_scatter(fsp_v, [fb + lane3], t * d0 + o0)
                    plsc.store_scatter(fsp_v, [fb + lane3 + 1], t * d1 + o1)
                    plsc.store_scatter(fsp_v, [fb + lane3 + 2], t * d2 + o2)

            pltpu.sync_copy(t_v, t_hbm.at[pl.ds(cbase * 64, CH * 64)])
            pltpu.sync_copy(fsp_v, fsp_hbm.at[pl.ds(cbase * 192, CH * 192)])

    return sc_stage


def kernel(rays, depth, density):
    n = depth.shape[0]
    depth_t = depth[:, :, 0].T
    dens_t = density[:, :, 0].T
    cdf_t, a_t, s_t = _tc_stage(depth_t, dens_t, n)
    cdfs = cdf_t.T.reshape(-1)
    alv = a_t.T.reshape(-1)
    slv = s_t.T.reshape(-1)
    raysf = rays.reshape(-1)
    u = jnp.linspace(0.0, 1.0, FN, dtype=jnp.float32)
    tflat, fspflat = _make_sc_stage(n)(cdfs, alv, slv, raysf, u)
    return (tflat.reshape(n, FN), fspflat.reshape(n, FN, 3))


# speedup vs baseline: 4.0703x; 4.0703x over previous
import functools
import jax
import jax.numpy as jnp
from jax import lax
from jax.experimental import pallas as pl
from jax.experimental.pallas import tpu as pltpu
from jax.experimental.pallas import tpu_sc as plsc

FN = 64
L = 64
RBLK = 256
NW = 32
CH = 128


def _tc_body(depth_ref, dens_ref, cdf_ref, a_ref, s_ref):
    R = depth_ref.shape[1]
    depth = depth_ref[...]
    dens = dens_ref[...]

    delta = jnp.concatenate(
        [depth[1:L] - depth[0:L - 1], jnp.full((1, R), 1e10, jnp.float32)], axis=0)
    neg_x = (-jnp.maximum(dens, 0.0)) * delta
    E = jnp.exp(neg_x)
    alpha = 1.0 - E
    terms = (1.0 - alpha) + 1e-10

    rows = [jnp.ones((1, R), jnp.float32)]
    for i in range(1, L):
        rows.append(rows[-1] * terms[i - 1:i])
    trans = jnp.concatenate(rows, axis=0)

    wp = alpha * trans + 1e-5

    P = wp[1:9]
    for k in range(1, 7):
        P = P + wp[1 + 8 * k: 9 + 8 * k]
    P = P + jnp.concatenate([wp[57:64], jnp.zeros((1, R), jnp.float32)], axis=0)
    W = (((P[0:1] + P[4:5]) + (P[2:3] + P[6:7]))
         + ((P[1:2] + P[5:6]) + (P[3:4] + P[7:8])))

    inv_w = 1.0 / W
    pdf = wp[1:L] * inv_w

    crows = [jnp.zeros((1, R), jnp.float32), pdf[0:1]]
    for i in range(1, L - 1):
        crows.append(crows[-1] + pdf[i:i + 1])
    cdf_s = jnp.concatenate(crows, axis=0)

    dn = cdf_s[1:L] - cdf_s[0:L - 1]
    dn = jnp.where(dn < 1e-5, jnp.ones_like(dn), dn)
    S_lv = (depth[1:L] - depth[0:L - 1]) / dn
    A_lv = depth[0:L - 1] - S_lv * cdf_s[0:L - 1]

    zrow = jnp.zeros((1, R), jnp.float32)
    cdf_ref[...] = cdf_s
    a_ref[...] = jnp.concatenate([A_lv, zrow], axis=0)
    s_ref[...] = jnp.concatenate([S_lv, zrow], axis=0)


def _tc_stage(depth_t, dens_t, n):
    grid = n // RBLK
    return pl.pallas_call(
        _tc_body,
        grid=(grid,),
        in_specs=[
            pl.BlockSpec((L, RBLK), lambda i: (0, i)),
            pl.BlockSpec((L, RBLK), lambda i: (0, i)),
        ],
        out_specs=[
            pl.BlockSpec((L, RBLK), lambda i: (0, i)),
            pl.BlockSpec((L, RBLK), lambda i: (0, i)),
            pl.BlockSpec((L, RBLK), lambda i: (0, i)),
        ],
        out_shape=[
            jax.ShapeDtypeStruct((L, n), jnp.float32),
            jax.ShapeDtypeStruct((L, n), jnp.float32),
            jax.ShapeDtypeStruct((L, n), jnp.float32),
        ],
        compiler_params=pltpu.CompilerParams(
            dimension_semantics=("arbitrary",),
        ),
    )(depth_t, dens_t)


_DG_DN = lax.GatherDimensionNumbers(
    offset_dims=(), collapsed_slice_dims=(0,), start_index_map=(0,))


def _dg(x, i):
    return lax.gather(x, i.reshape(16, 1), _DG_DN, (1,),
                      mode=lax.GatherScatterMode.PROMISE_IN_BOUNDS)


def _make_sc_stage(n):
    mesh = plsc.VectorSubcoreMesh(core_axis_name="c", subcore_axis_name="s")
    per_w = n // NW

    @functools.partial(
        pl.kernel, mesh=mesh,
        out_type=[
            jax.ShapeDtypeStruct((n * 64,), jnp.float32),
            jax.ShapeDtypeStruct((3 * n * 64,), jnp.float32),
        ],
        scratch_types=[
            pltpu.VMEM((64,), jnp.float32),
            pltpu.VMEM((CH * 64,), jnp.float32),
            pltpu.VMEM((CH * 64,), jnp.float32),
            pltpu.VMEM((CH * 64,), jnp.float32),
            pltpu.VMEM((CH * 6 + 16,), jnp.float32),
            pltpu.VMEM((CH * 64,), jnp.float32),
            pltpu.VMEM((3 * CH * 64,), jnp.float32),
        ],
    )
    def sc_stage(cdfs_hbm, alv_hbm, slv_hbm, rays_hbm, u_hbm,
                 t_hbm, fsp_hbm,
                 u_v, cdf_v, alv_v, slv_v, rays_v, t_v, fsp_v):
        wid = lax.axis_index("s") * 2 + lax.axis_index("c")
        base_ray = wid * per_w
        pltpu.sync_copy(u_hbm, u_v)

        @pl.loop(0, per_w // CH)
        def _chunk(ci):
            cbase = base_ray + ci * CH
            pltpu.sync_copy(cdfs_hbm.at[pl.ds(cbase * 64, CH * 64)], cdf_v)
            pltpu.sync_copy(alv_hbm.at[pl.ds(cbase * 64, CH * 64)], alv_v)
            pltpu.sync_copy(slv_hbm.at[pl.ds(cbase * 64, CH * 64)], slv_v)
            pltpu.sync_copy(rays_hbm.at[pl.ds(cbase * 6, CH * 6)],
                            rays_v.at[pl.ds(0, CH * 6)])

            @pl.loop(0, CH)
            def _ray(r):
                b = r * 64
                cks = [cdf_v[pl.ds(b + 16 * j, 16)] for j in range(4)]
                aks = [alv_v[pl.ds(b + 16 * j, 16)] for j in range(4)]
                sks = [slv_v[pl.ds(b + 16 * j, 16)] for j in range(4)]
                last = jnp.full((16,), 15, jnp.int32)
                c15 = _dg(cks[0], last)
                c31 = _dg(cks[1], last)
                c47 = _dg(cks[2], last)
                c63 = _dg(cks[3], last)

                rv = rays_v[pl.ds(r * 6, 16)]
                o0 = _dg(rv, jnp.full((16,), 0, jnp.int32))
                o1 = _dg(rv, jnp.full((16,), 1, jnp.int32))
                o2 = _dg(rv, jnp.full((16,), 2, jnp.int32))
                d0 = _dg(rv, jnp.full((16,), 3, jnp.int32))
                d1 = _dg(rv, jnp.full((16,), 4, jnp.int32))
                d2 = _dg(rv, jnp.full((16,), 5, jnp.int32))

                def val4(vregs, hi_, lo_):
                    gs = [_dg(v, lo_) for v in vregs]
                    return jnp.where(
                        hi_ == 0, gs[0],
                        jnp.where(hi_ == 1, gs[1],
                                  jnp.where(hi_ == 2, gs[2], gs[3])))

                for k in range(4):
                    uu = u_v[pl.ds(16 * k, 16)]
                    zero16 = jnp.zeros((16,), jnp.int32)
                    one16 = jnp.full((16,), 1, jnp.int32)
                    hi = (jnp.where(c15 < uu, one16, zero16)
                          + jnp.where(c31 < uu, one16, zero16)
                          + jnp.where(c47 < uu, one16, zero16))
                    lo = jnp.zeros((16,), jnp.int32)
                    for s in (8, 4, 2, 1):
                        g = val4(cks, hi, lo + (s - 1))
                        lo = jnp.where(g < uu, lo + s, lo)
                    pos = hi * 16 + lo
                    m64 = c63 < uu
                    lvl = jnp.maximum(pos, one16) - one16
                    idx = jnp.where(m64, zero16, lvl)
                    ihi = (jnp.where(idx >= 16, one16, zero16)
                           + jnp.where(idx >= 32, one16, zero16)
                           + jnp.where(idx >= 48, one16, zero16))
                    ilo = idx - ihi * 16
                    A = val4(aks, ihi, ilo)
                    S = val4(sks, ihi, ilo)
                    t = A + S * uu
                    t_v[pl.ds(b + 16 * k, 16)] = t
                    fsp_v[pl.ds(b + 16 * k, 16)] = t * d0 + o0
                    fsp_v[pl.ds(CH * 64 + b + 16 * k, 16)] = t * d1 + o1
                    fsp_v[pl.ds(2 * CH * 64 + b + 16 * k, 16)] = t * d2 + o2

            pltpu.sync_copy(t_v, t_hbm.at[pl.ds(cbase * 64, CH * 64)])
            for d in range(3):
                pltpu.sync_copy(
                    fsp_v.at[pl.ds(d * CH * 64, CH * 64)],
                    fsp_hbm.at[pl.ds(d * n * 64 + cbase * 64, CH * 64)])

    return sc_stage


def kernel(rays, depth, density):
    n = depth.shape[0]
    depth_t = depth[:, :, 0].T
    dens_t = density[:, :, 0].T
    cdf_t, a_t, s_t = _tc_stage(depth_t, dens_t, n)
    cdfs = cdf_t.T.reshape(-1)
    alv = a_t.T.reshape(-1)
    slv = s_t.T.reshape(-1)
    raysf = rays.reshape(-1)
    u = jnp.linspace(0.0, 1.0, FN, dtype=jnp.float32)
    tflat, fspflat = _make_sc_stage(n)(cdfs, alv, slv, raysf, u)
    fsp = jnp.transpose(fspflat.reshape(3, n, FN), (1, 2, 0))
    return (tflat.reshape(n, FN), fsp)
